# baseline (speedup 1.0000x reference)
import jax
import jax.numpy as jnp
from jax import lax
from jax.experimental import pallas as pl
from jax.experimental.pallas import tpu as pltpu
from jax.experimental.pallas import tpu_sc as plsc

BATCH = 4096
SEQ = 200
EMB_DIM = 64
PROJ_DIM = 128
NBINS = 64

NC = 2
NS = 16
L = 16
NW = NC * NS
ROWS_PER_W = BATCH // NW
GROUPS = ROWS_PER_W // L
SEQ_PITCH = 201
NBINS_PITCH = 65


def _sc_counts_body(ids_hbm, countsT_hbm, ids_v, counts_v, sem):
    wid = lax.axis_index("s") * NC + lax.axis_index("c")
    lane = lax.iota(jnp.int32, L)
    ones = jnp.ones((L,), jnp.float32)
    r0 = wid * ROWS_PER_W

    cp = pltpu.async_copy(ids_hbm.at[pl.ds(r0, ROWS_PER_W), :], ids_v, sem)
    zeros = jnp.zeros((L,), jnp.float32)
    for v in range(NBINS):
        for j in range(GROUPS):
            counts_v[v, pl.ds(j * L, L)] = zeros
    cp.wait()

    cols = [lane + k * L for k in range(GROUPS)]

    @plsc.parallel_loop(0, SEQ)
    def s_body(s0):
        svec = lane + jnp.full((L,), s0, jnp.int32)
        svec = jnp.where(svec >= SEQ, svec - SEQ, svec)
        for k in range(GROUPS):
            ids16 = plsc.load_gather(ids_v, [cols[k], svec])
            plsc.addupdate_scatter(counts_v, [ids16, cols[k]], ones)

    pltpu.sync_copy(counts_v, countsT_hbm.at[:, pl.ds(r0, ROWS_PER_W)])


def _sc_counts(input_ids):
    mesh = plsc.VectorSubcoreMesh(core_axis_name="c", subcore_axis_name="s")
    return pl.kernel(
        _sc_counts_body,
        out_type=jax.ShapeDtypeStruct((NBINS, BATCH), jnp.float32),
        mesh=mesh,
        scratch_types=[
            pltpu.VMEM((ROWS_PER_W, SEQ), jnp.int32),
            pltpu.VMEM((NBINS, ROWS_PER_W), jnp.float32),
            pltpu.SemaphoreType.DMA,
        ],
        compiler_params=pltpu.CompilerParams(
            needs_layout_passes=False, use_tc_tiling_on_sc=True
        ),
    )(input_ids)


BB = 2048


def _tc_proj_body(countsT_ref, embT_ref, wT_ref, b_ref, idf_ref, out_ref):
    countsT = countsT_ref[...]
    idfc = idf_ref[...]
    wT = countsT * idfc
    nrm2 = jnp.sum(wT * idfc, axis=0, keepdims=True)
    denom = jnp.maximum(jnp.sqrt(nrm2), 1e-12)
    wnT = wT / denom

    sembT = jnp.dot(embT_ref[...], wnT, preferred_element_type=jnp.float32)
    outT = jnp.dot(wT_ref[...], sembT, preferred_element_type=jnp.float32)
    out_ref[...] = outT.T + b_ref[...]


def kernel(input_ids, emb_table, W, b, idf_table):
    countsT = _sc_counts(input_ids)
    embT = emb_table[:NBINS].T
    WT = W.T
    idfc = idf_table.reshape(NBINS, 1)
    b2d = b.reshape(1, PROJ_DIM)
    grid = (BATCH // BB,)
    return pl.pallas_call(
        _tc_proj_body,
        grid=grid,
        in_specs=[
            pl.BlockSpec((NBINS, BB), lambda i: (0, i)),
            pl.BlockSpec((EMB_DIM, NBINS), lambda i: (0, 0)),
            pl.BlockSpec((PROJ_DIM, EMB_DIM), lambda i: (0, 0)),
            pl.BlockSpec((1, PROJ_DIM), lambda i: (0, 0)),
            pl.BlockSpec((NBINS, 1), lambda i: (0, 0)),
        ],
        out_specs=pl.BlockSpec((BB, PROJ_DIM), lambda i: (i, 0)),
        out_shape=jax.ShapeDtypeStruct((BATCH, PROJ_DIM), jnp.float32),
    )(countsT, embT, WT, b2d, idfc)


# speedup vs baseline: 1.0829x; 1.0829x over previous
import jax
import jax.numpy as jnp
from jax import lax
from jax.experimental import pallas as pl
from jax.experimental.pallas import tpu as pltpu
from jax.experimental.pallas import tpu_sc as plsc

BATCH = 4096
SEQ = 200
EMB_DIM = 64
PROJ_DIM = 128
NBINS = 64

NC = 2
NS = 16
L = 16
NW = NC * NS
ROWS_PER_W = BATCH // NW
GROUPS = ROWS_PER_W // L
SEQ_PITCH = 201
NBINS_PITCH = 65


def _sc_counts_body(idsT_hbm, countsT_hbm, ids_v, counts_v, sem):
    wid = lax.axis_index("s") * NC + lax.axis_index("c")
    lane = lax.iota(jnp.int32, L)
    ones = jnp.ones((L,), jnp.float32)
    r0 = wid * ROWS_PER_W

    cp = pltpu.async_copy(idsT_hbm.at[:, pl.ds(r0, ROWS_PER_W)], ids_v, sem)
    zeros = jnp.zeros((L,), jnp.float32)
    for v in range(NBINS):
        for j in range(GROUPS):
            counts_v[v, pl.ds(j * L, L)] = zeros
    cp.wait()

    cols = [lane + k * L for k in range(GROUPS)]

    @plsc.parallel_loop(0, SEQ)
    def s_body(s):
        svec = jnp.full((L,), s, jnp.int32)
        for k in range(GROUPS):
            ids16 = plsc.load_gather(ids_v, [svec, cols[k]])
            plsc.addupdate_scatter(counts_v, [ids16, cols[k]], ones)

    pltpu.sync_copy(counts_v, countsT_hbm.at[:, pl.ds(r0, ROWS_PER_W)])


def _sc_counts(ids_t):
    mesh = plsc.VectorSubcoreMesh(core_axis_name="c", subcore_axis_name="s")
    return pl.kernel(
        _sc_counts_body,
        out_type=jax.ShapeDtypeStruct((NBINS, BATCH), jnp.float32),
        mesh=mesh,
        scratch_types=[
            pltpu.VMEM((SEQ, ROWS_PER_W), jnp.int32),
            pltpu.VMEM((NBINS, ROWS_PER_W), jnp.float32),
            pltpu.SemaphoreType.DMA,
        ],
        compiler_params=pltpu.CompilerParams(
            needs_layout_passes=False, use_tc_tiling_on_sc=True
        ),
    )(ids_t)


BB = 2048


def _tc_proj_body(countsT_ref, embT_ref, wT_ref, b_ref, idf_ref, out_ref):
    countsT = countsT_ref[...]
    idfc = idf_ref[...]
    wT = countsT * idfc
    nrm2 = jnp.sum(wT * idfc, axis=0, keepdims=True)
    denom = jnp.maximum(jnp.sqrt(nrm2), 1e-12)
    wnT = wT / denom

    sembT = jnp.dot(embT_ref[...], wnT, preferred_element_type=jnp.float32)
    outT = jnp.dot(wT_ref[...], sembT, preferred_element_type=jnp.float32)
    out_ref[...] = outT.T + b_ref[...]


def kernel(input_ids, emb_table, W, b, idf_table):
    countsT = _sc_counts(input_ids.T)
    embT = emb_table[:NBINS].T
    WT = W.T
    idfc = idf_table.reshape(NBINS, 1)
    b2d = b.reshape(1, PROJ_DIM)
    grid = (BATCH // BB,)
    return pl.pallas_call(
        _tc_proj_body,
        grid=grid,
        in_specs=[
            pl.BlockSpec((NBINS, BB), lambda i: (0, i)),
            pl.BlockSpec((EMB_DIM, NBINS), lambda i: (0, 0)),
            pl.BlockSpec((PROJ_DIM, EMB_DIM), lambda i: (0, 0)),
            pl.BlockSpec((1, PROJ_DIM), lambda i: (0, 0)),
            pl.BlockSpec((NBINS, 1), lambda i: (0, 0)),
        ],
        out_specs=pl.BlockSpec((BB, PROJ_DIM), lambda i: (i, 0)),
        out_shape=jax.ShapeDtypeStruct((BATCH, PROJ_DIM), jnp.float32),
    )(countsT, embT, WT, b2d, idfc)
